# baseline (speedup 1.0000x reference)
import functools

import jax
import jax.numpy as jnp
from jax import lax
from jax.experimental import pallas as pl
from jax.experimental.pallas import tpu as pltpu
from jax.experimental.pallas import tpu_sc as plsc

_L = 16


@functools.lru_cache(maxsize=None)
def _build_sc_lookup(batch: int, vocab: int):
    info = plsc.get_sparse_core_info()
    num_workers = info.num_cores * info.num_subcores
    b_per_w = batch // num_workers
    assert b_per_w % _L == 0 and (b_per_w * 4) % 8 == 0
    n_scatter = -(-vocab // _L)
    v_pad = n_scatter * _L
    n_gather = b_per_w // _L

    mesh = plsc.VectorSubcoreMesh(core_axis_name="c", subcore_axis_name="s")

    @functools.partial(
        pl.kernel,
        mesh=mesh,
        out_type=jax.ShapeDtypeStruct((batch,), jnp.int32),
        compiler_params=pltpu.CompilerParams(
            needs_layout_passes=False,
            skip_device_barrier=True,
            disable_bounds_checks=True,
            disable_semaphore_checks=True,
        ),
        scratch_types=[
            pltpu.VMEM((v_pad,), jnp.int32),
            pltpu.VMEM((v_pad,), jnp.int32),
            pltpu.VMEM((b_per_w,), jnp.int32),
            pltpu.VMEM((b_per_w,), jnp.int32),
            pltpu.SemaphoreType.DMA,
            pltpu.SemaphoreType.DMA,
        ],
    )
    def lookup(name_hbm, i2n_hbm, out_hbm, i2n_v, table_v, name_v, out_v,
               sem_a, sem_b):
        wid = lax.axis_index("s") * info.num_cores + lax.axis_index("c")
        base = wid * b_per_w
        cp_i2n = pltpu.async_copy(i2n_hbm, i2n_v.at[pl.ds(0, vocab)], sem_a)
        cp_name = pltpu.async_copy(
            name_hbm.at[pl.ds(base, b_per_w)], name_v, sem_b
        )
        cp_i2n.wait()

        full = (n_scatter - 1) * _L

        @plsc.parallel_loop(0, full, _L, unroll=4)
        def _scatter(i):
            keys = i2n_v[pl.ds(i, _L)]
            plsc.store_scatter(table_v, [keys], lax.iota(jnp.int32, _L) + i)

        tail_pos = lax.iota(jnp.int32, _L) + full
        tail_keys = i2n_v[pl.ds(full, _L)]
        plsc.store_scatter(table_v, [tail_keys], tail_pos,
                           mask=tail_pos < vocab)
        cp_name.wait()

        @plsc.parallel_loop(0, b_per_w, _L, unroll=4)
        def _gather(i):
            q = name_v[pl.ds(i, _L)]
            out_v[pl.ds(i, _L)] = plsc.load_gather(table_v, [q])

        pltpu.sync_copy(out_v, out_hbm.at[pl.ds(base, b_per_w)])

    return lookup


def kernel(name, idx_to_name):
    return _build_sc_lookup(name.shape[0], idx_to_name.shape[0])(
        name, idx_to_name
    )


# speedup vs baseline: 1.0030x; 1.0030x over previous
import functools

import jax
import jax.numpy as jnp
from jax import lax
from jax.experimental import pallas as pl
from jax.experimental.pallas import tpu as pltpu
from jax.experimental.pallas import tpu_sc as plsc

_L = 16


@functools.lru_cache(maxsize=None)
def _build_sc_lookup(batch: int, vocab: int):
    info = plsc.get_sparse_core_info()
    num_workers = info.num_cores * info.num_subcores
    b_per_w = batch // num_workers
    assert b_per_w % _L == 0 and (b_per_w * 4) % 8 == 0
    n_scatter = -(-vocab // _L)
    v_pad = n_scatter * _L
    n_gather = b_per_w // _L

    mesh = plsc.VectorSubcoreMesh(core_axis_name="c", subcore_axis_name="s")

    @functools.partial(
        pl.kernel,
        mesh=mesh,
        out_type=jax.ShapeDtypeStruct((batch,), jnp.int32),
        compiler_params=pltpu.CompilerParams(needs_layout_passes=False),
        scratch_types=[
            pltpu.VMEM((v_pad,), jnp.int32),
            pltpu.VMEM((v_pad,), jnp.int32),
            pltpu.VMEM((b_per_w,), jnp.int32),
            pltpu.VMEM((b_per_w,), jnp.int32),
            pltpu.SemaphoreType.DMA,
            pltpu.SemaphoreType.DMA,
        ],
    )
    def lookup(name_hbm, i2n_hbm, out_hbm, i2n_v, table_v, name_v, out_v,
               sem_a, sem_b):
        wid = lax.axis_index("s") * info.num_cores + lax.axis_index("c")
        base = wid * b_per_w
        cp_i2n = pltpu.async_copy(i2n_hbm, i2n_v.at[pl.ds(0, vocab)], sem_a)
        cp_name = pltpu.async_copy(
            name_hbm.at[pl.ds(base, b_per_w)], name_v, sem_b
        )
        cp_i2n.wait()

        full = (n_scatter - 1) * _L

        @plsc.parallel_loop(0, full, _L, unroll=2)
        def _scatter(i):
            keys = i2n_v[pl.ds(i, _L)]
            plsc.store_scatter(table_v, [keys], lax.iota(jnp.int32, _L) + i)

        tail_pos = lax.iota(jnp.int32, _L) + full
        tail_keys = i2n_v[pl.ds(full, _L)]
        plsc.store_scatter(table_v, [tail_keys], tail_pos,
                           mask=tail_pos < vocab)
        cp_name.wait()

        @plsc.parallel_loop(0, b_per_w, _L, unroll=2)
        def _gather(i):
            q = name_v[pl.ds(i, _L)]
            out_v[pl.ds(i, _L)] = plsc.load_gather(table_v, [q])

        pltpu.sync_copy(out_v, out_hbm.at[pl.ds(base, b_per_w)])

    return lookup


def kernel(name, idx_to_name):
    return _build_sc_lookup(name.shape[0], idx_to_name.shape[0])(
        name, idx_to_name
    )
